# Initial kernel scaffold; baseline (speedup 1.0000x reference)
import jax
import jax.numpy as jnp
from jax.experimental import pallas as pl


def kernel(indices, table):
    raise NotImplementedError("write your pallas kernel here")


# speedup vs baseline: 4.9438x; 4.9438x over previous
import functools

import jax
import jax.numpy as jnp
from jax import lax
from jax.experimental import pallas as pl
from jax.experimental.pallas import tpu as pltpu
from jax.experimental.pallas import tpu_sc as plsc


def _emb_call(N, D, NC, NS):
    NW = NC * NS
    n_per_w = N // NW
    C = 128
    K = 16
    BLK = K * C
    n_blk = n_per_w // BLK

    mesh = plsc.VectorSubcoreMesh(core_axis_name="c", subcore_axis_name="s",
                                  num_cores=NC, num_subcores=NS)

    @functools.partial(
        pl.kernel,
        out_type=jax.ShapeDtypeStruct((N, D), jnp.float32),
        mesh=mesh,
        scratch_types=[
            pltpu.VMEM((K, C), jnp.int32),
            pltpu.VMEM((BLK, D), jnp.float32),
            pltpu.SemaphoreType.DMA,
        ],
        compiler_params=pltpu.CompilerParams(use_tc_tiling_on_sc=False),
    )
    def emb(idx_hbm, table_hbm, out_hbm, idx_v, rows_v, sem):
        wid = lax.axis_index("s") * NC + lax.axis_index("c")
        base = wid * n_per_w

        def body(i, carry):
            b = pl.multiple_of(base + i * BLK, BLK)
            pltpu.sync_copy(idx_hbm.at[pl.ds(pl.multiple_of(b // C, K), K)],
                            idx_v)
            copies = [
                pltpu.async_copy(table_hbm.at[idx_v.at[j]],
                                 rows_v.at[pl.ds(j * C, C)], sem)
                for j in range(K)
            ]
            for cp in copies:
                cp.wait()
            pltpu.sync_copy(rows_v, out_hbm.at[pl.ds(b, BLK)])
            return carry

        lax.fori_loop(0, n_blk, body, 0)

    return emb


def kernel(indices, table):
    B, L = indices.shape
    V, D = table.shape
    N = B * L
    info = plsc.get_sparse_core_info()
    NC, NS = info.num_cores, info.num_subcores
    idx2d = indices.reshape(N // 128, 128).astype(jnp.int32)
    out = _emb_call(N, D, NC, NS)(idx2d, table)
    return out.reshape(B, L, D)


# speedup vs baseline: 4.9552x; 1.0023x over previous
import functools

import jax
import jax.numpy as jnp
from jax import lax
from jax.experimental import pallas as pl
from jax.experimental.pallas import tpu as pltpu
from jax.experimental.pallas import tpu_sc as plsc


def _emb_call(N, D, NC, NS):
    NW = NC * NS
    n_per_w = N // NW
    C = 128
    K = 8
    BLK = K * C
    NBUF = 2
    n_blk = n_per_w // BLK
    assert n_per_w % BLK == 0 and n_blk % NBUF == 0

    mesh = plsc.VectorSubcoreMesh(core_axis_name="c", subcore_axis_name="s",
                                  num_cores=NC, num_subcores=NS)

    @functools.partial(
        pl.kernel,
        out_type=jax.ShapeDtypeStruct((N, D), jnp.float32),
        mesh=mesh,
        scratch_types=[
            pltpu.VMEM((NBUF, K, C), jnp.int32),
            pltpu.VMEM((NBUF, BLK, D), jnp.float32),
            pltpu.SemaphoreType.DMA,
            pltpu.SemaphoreType.DMA,
            pltpu.SemaphoreType.DMA,
        ],
        compiler_params=pltpu.CompilerParams(use_tc_tiling_on_sc=False),
    )
    def emb(idx_hbm, table_hbm, out_hbm, idx_v, rows_v, gsem, osem0, osem1):
        osems = (osem0, osem1)
        wid = lax.axis_index("s") * NC + lax.axis_index("c")
        base = wid * n_per_w

        def load_gather(i, p):
            b = pl.multiple_of(base + i * BLK, BLK)
            pltpu.sync_copy(idx_hbm.at[pl.ds(pl.multiple_of(b // C, K), K)],
                            idx_v.at[p])
            cps = [
                pltpu.async_copy(table_hbm.at[idx_v.at[p, j]],
                                 rows_v.at[p, pl.ds(j * C, C)], gsem)
                for j in range(K)
            ]
            for cp in cps:
                cp.wait()

        def start_store(i, p):
            b = pl.multiple_of(base + i * BLK, BLK)
            pltpu.async_copy(rows_v.at[p], out_hbm.at[pl.ds(b, BLK)], osems[p])

        def wait_store(p):
            pltpu.make_async_copy(out_hbm.at[pl.ds(0, BLK)], rows_v.at[p],
                                  osems[p]).wait()

        for p in range(NBUF):
            load_gather(p, p)
            start_store(p, p)

        def body(g, carry):
            for p in range(NBUF):
                i = g * NBUF + p
                wait_store(p)
                load_gather(i, p)
                start_store(i, p)
            return carry

        lax.fori_loop(1, n_blk // NBUF, body, 0)

        for p in range(NBUF):
            wait_store(p)

    return emb


def kernel(indices, table):
    B, L = indices.shape
    V, D = table.shape
    N = B * L
    info = plsc.get_sparse_core_info()
    NC, NS = info.num_cores, info.num_subcores
    idx2d = indices.reshape(N // 128, 128).astype(jnp.int32)
    out = _emb_call(N, D, NC, NS)(idx2d, table)
    return out.reshape(B, L, D)


# speedup vs baseline: 5.0474x; 1.0186x over previous
import functools

import jax
import jax.numpy as jnp
from jax import lax
from jax.experimental import pallas as pl
from jax.experimental.pallas import tpu as pltpu
from jax.experimental.pallas import tpu_sc as plsc


def _emb_call(N, D, NC, NS):
    NW = NC * NS
    n_per_w = N // NW
    C = 128
    K = 8
    BLK = K * C
    NBUF = 3
    n_blk = n_per_w // BLK
    assert n_per_w % BLK == 0 and n_blk >= NBUF + 1

    mesh = plsc.VectorSubcoreMesh(core_axis_name="c", subcore_axis_name="s",
                                  num_cores=NC, num_subcores=NS)

    @functools.partial(
        pl.kernel,
        out_type=jax.ShapeDtypeStruct((N, D), jnp.float32),
        mesh=mesh,
        scratch_types=[
            pltpu.VMEM((NBUF, K, C), jnp.int32),
            pltpu.VMEM((NBUF, BLK, D), jnp.float32),
            [pltpu.SemaphoreType.DMA] * NBUF,
            [pltpu.SemaphoreType.DMA] * NBUF,
        ],
        compiler_params=pltpu.CompilerParams(use_tc_tiling_on_sc=False),
    )
    def emb(idx_hbm, table_hbm, out_hbm, idx_v, rows_v, gsems, osems):
        wid = lax.axis_index("s") * NC + lax.axis_index("c")
        base = wid * n_per_w

        def fire(i, p):
            b = pl.multiple_of(base + i * BLK, BLK)
            pltpu.sync_copy(idx_hbm.at[pl.ds(pl.multiple_of(b // C, K), K)],
                            idx_v.at[p])
            for j in range(K):
                pltpu.async_copy(table_hbm.at[idx_v.at[p, j]],
                                 rows_v.at[p, pl.ds(j * C, C)], gsems[p])

        def drain_gathers(p):
            pltpu.make_async_copy(out_hbm.at[pl.ds(0, BLK)], rows_v.at[p],
                                  gsems[p]).wait()

        def start_store(i, p):
            b = pl.multiple_of(base + i * BLK, BLK)
            pltpu.async_copy(rows_v.at[p], out_hbm.at[pl.ds(b, BLK)],
                             osems[p])

        def wait_store(p):
            pltpu.make_async_copy(out_hbm.at[pl.ds(0, BLK)], rows_v.at[p],
                                  osems[p]).wait()

        fire(0, 0)
        for i in range(1, NBUF):
            fire(i, i)
            drain_gathers(i - 1)
            start_store(i - 1, i - 1)

        n_loop = (n_blk - NBUF) // NBUF

        def body(g, carry):
            for t in range(NBUF):
                i = g * NBUF + t
                wait_store(t)
                fire(i, t)
                q = (t + NBUF - 1) % NBUF
                drain_gathers(q)
                start_store(i - 1, q)
            return carry

        lax.fori_loop(1, n_loop + 1, body, 0)

        for r in range(NBUF * (n_loop + 1), n_blk):
            t = r % NBUF
            wait_store(t)
            fire(r, t)
            q = (t + NBUF - 1) % NBUF
            drain_gathers(q)
            start_store(r - 1, q)

        last = n_blk - 1
        drain_gathers(last % NBUF)
        start_store(last, last % NBUF)
        for p in range(NBUF):
            wait_store(p)

    return emb


def kernel(indices, table):
    B, L = indices.shape
    V, D = table.shape
    N = B * L
    info = plsc.get_sparse_core_info()
    NC, NS = info.num_cores, info.num_subcores
    idx2d = indices.reshape(N // 128, 128).astype(jnp.int32)
    out = _emb_call(N, D, NC, NS)(idx2d, table)
    return out.reshape(B, L, D)
